# Initial kernel scaffold; baseline (speedup 1.0000x reference)
import functools

import jax
import jax.numpy as jnp
from jax import lax
from jax.experimental import pallas as pl
from jax.experimental.pallas import tpu as pltpu
from jax.experimental.pallas import tpu_sc as plsc

LANES = 16


@functools.cache
def _build_pool_kernel(B, L, D, CH):
    info = plsc.get_sparse_core_info()
    NC, NS = info.num_cores, info.num_subcores
    NW = NC * NS
    RPW = B // NW
    NCH = RPW // CH
    GI = L // 2
    G = 2 * CH
    inv_l = 1.0 / L

    mesh = plsc.VectorSubcoreMesh(core_axis_name="c", subcore_axis_name="s")

    @functools.partial(
        pl.kernel,
        mesh=mesh,
        out_type=jax.ShapeDtypeStruct((B, D), jnp.float32),
        scratch_types=[
            pltpu.VMEM((G, GI), jnp.int32),
            pltpu.VMEM((G, GI), jnp.int32),
            pltpu.VMEM((CH * L, D), jnp.float32),
            pltpu.VMEM((CH * L, D), jnp.float32),
            pltpu.VMEM((RPW, D), jnp.float32),
            pltpu.SemaphoreType.DMA,
            pltpu.SemaphoreType.DMA,
        ],
    )
    def body(ids_hbm, table_hbm, out_hbm,
             idx_a, idx_b, rows_a, rows_b, out_v, sem_a, sem_b):
        wid = lax.axis_index("s") * NC + lax.axis_index("c")
        wbase = wid * RPW

        def copy_idx(c, idxv):
            start = 2 * (wbase + c * CH)
            pltpu.sync_copy(ids_hbm.at[pl.ds(start, G)], idxv)

        def fire(idxv, rowsv, sem):
            for j in range(G):
                pltpu.async_copy(table_hbm.at[idxv.at[j]],
                                 rowsv.at[pl.ds(j * GI, GI)], sem)

        def drain(idxv, rowsv, sem):
            for j in range(G):
                pltpu.make_async_copy(table_hbm.at[idxv.at[j]],
                                      rowsv.at[pl.ds(j * GI, GI)], sem).wait()

        def accum(c, rowsv):
            zero = jnp.zeros((LANES,), jnp.float32)
            for o in range(CH):
                def inner(r, carry, _o=o):
                    a0, a1, b0, b1 = carry
                    r0 = _o * L + 2 * r
                    a0 = a0 + rowsv[r0, pl.ds(0, LANES)]
                    a1 = a1 + rowsv[r0, pl.ds(LANES, LANES)]
                    b0 = b0 + rowsv[r0 + 1, pl.ds(0, LANES)]
                    b1 = b1 + rowsv[r0 + 1, pl.ds(LANES, LANES)]
                    return a0, a1, b0, b1
                a0, a1, b0, b1 = lax.fori_loop(
                    0, L // 2, inner, (zero, zero, zero, zero))
                row = c * CH + o
                out_v[row, pl.ds(0, LANES)] = (a0 + b0) * inv_l
                out_v[row, pl.ds(LANES, LANES)] = (a1 + b1) * inv_l

        copy_idx(0, idx_a)
        fire(idx_a, rows_a, sem_a)

        def step2(g, carry):
            c0 = 2 * g
            copy_idx(c0 + 1, idx_b)
            fire(idx_b, rows_b, sem_b)
            drain(idx_a, rows_a, sem_a)
            accum(c0, rows_a)

            @pl.when(c0 + 2 < NCH)
            def _():
                copy_idx(c0 + 2, idx_a)
                fire(idx_a, rows_a, sem_a)

            drain(idx_b, rows_b, sem_b)
            accum(c0 + 1, rows_b)
            return carry

        lax.fori_loop(0, NCH // 2, step2, 0)
        pltpu.sync_copy(out_v, out_hbm.at[pl.ds(wbase, RPW)])

    return body


def kernel(token_ids, token_emb_weight, null_context):
    B, L = token_ids.shape
    V, D = token_emb_weight.shape
    ids = token_ids.astype(jnp.int32).reshape(2 * B, L // 2)
    pool = _build_pool_kernel(B, L, D, CH=8)
    return pool(ids, token_emb_weight)


# speedup vs baseline: 15.1027x; 15.1027x over previous
import functools

import jax
import jax.numpy as jnp
from jax import lax
from jax.experimental import pallas as pl
from jax.experimental.pallas import tpu as pltpu
from jax.experimental.pallas import tpu_sc as plsc

LANES = 16


@functools.cache
def _build_pool_kernel(B, L, D, CH):
    info = plsc.get_sparse_core_info()
    NC, NS = info.num_cores, info.num_subcores
    NW = NC * NS
    RPW = B // NW
    NCH = RPW // CH
    GI = L // 2
    G = 2 * CH
    inv_l = 1.0 / L

    mesh = plsc.VectorSubcoreMesh(core_axis_name="c", subcore_axis_name="s")

    @functools.partial(
        pl.kernel,
        mesh=mesh,
        out_type=jax.ShapeDtypeStruct((B, D), jnp.float32),
        compiler_params=pltpu.CompilerParams(use_tc_tiling_on_sc=False),
        scratch_types=[
            pltpu.VMEM((G, GI), jnp.int32),
            pltpu.VMEM((G, GI), jnp.int32),
            pltpu.VMEM((CH * L, D), jnp.float32),
            pltpu.VMEM((CH * L, D), jnp.float32),
            pltpu.VMEM((RPW, D), jnp.float32),
            pltpu.SemaphoreType.DMA,
            pltpu.SemaphoreType.DMA,
        ],
    )
    def body(ids_hbm, table_hbm, out_hbm,
             idx_a, idx_b, rows_a, rows_b, out_v, sem_a, sem_b):
        wid = lax.axis_index("s") * NC + lax.axis_index("c")
        wbase = wid * RPW

        def copy_idx(c, idxv):
            start = 2 * (wbase + c * CH)
            pltpu.sync_copy(ids_hbm.at[pl.ds(start, G)], idxv)

        def fire(idxv, rowsv, sem):
            for j in range(G):
                pltpu.async_copy(table_hbm.at[idxv.at[j]],
                                 rowsv.at[pl.ds(j * GI, GI)], sem)

        def drain(idxv, rowsv, sem):
            for j in range(G):
                pltpu.make_async_copy(table_hbm.at[idxv.at[j]],
                                      rowsv.at[pl.ds(j * GI, GI)], sem).wait()

        def accum(c, rowsv):
            zero = jnp.zeros((LANES,), jnp.float32)
            for o in range(CH):
                def inner(r, carry, _o=o):
                    a0, a1, b0, b1 = carry
                    r0 = _o * L + 2 * r
                    a0 = a0 + rowsv[r0, pl.ds(0, LANES)]
                    a1 = a1 + rowsv[r0, pl.ds(LANES, LANES)]
                    b0 = b0 + rowsv[r0 + 1, pl.ds(0, LANES)]
                    b1 = b1 + rowsv[r0 + 1, pl.ds(LANES, LANES)]
                    return a0, a1, b0, b1
                a0, a1, b0, b1 = lax.fori_loop(
                    0, L // 2, inner, (zero, zero, zero, zero))
                row = c * CH + o
                out_v[row, pl.ds(0, LANES)] = (a0 + b0) * inv_l
                out_v[row, pl.ds(LANES, LANES)] = (a1 + b1) * inv_l

        copy_idx(0, idx_a)
        fire(idx_a, rows_a, sem_a)

        def step2(g, carry):
            c0 = 2 * g
            copy_idx(c0 + 1, idx_b)
            fire(idx_b, rows_b, sem_b)
            drain(idx_a, rows_a, sem_a)
            accum(c0, rows_a)

            @pl.when(c0 + 2 < NCH)
            def _():
                copy_idx(c0 + 2, idx_a)
                fire(idx_a, rows_a, sem_a)

            drain(idx_b, rows_b, sem_b)
            accum(c0 + 1, rows_b)
            return carry

        lax.fori_loop(0, NCH // 2, step2, 0)
        pltpu.sync_copy(out_v, out_hbm.at[pl.ds(wbase, RPW)])

    return body


def kernel(token_ids, token_emb_weight, null_context):
    B, L = token_ids.shape
    V, D = token_emb_weight.shape
    ids = token_ids.astype(jnp.int32).reshape(2 * B, L // 2)
    pool = _build_pool_kernel(B, L, D, CH=8)
    return pool(ids, token_emb_weight)
